# Initial kernel scaffold; baseline (speedup 1.0000x reference)
import jax
import jax.numpy as jnp
from jax.experimental import pallas as pl


def kernel(x, edge_index, edge_attr, batch, global_feature, params):
    raise NotImplementedError("write your pallas kernel here")


# speedup vs baseline: 2.8439x; 2.8439x over previous
import functools

import jax
import jax.numpy as jnp
from jax import lax
from jax.experimental import pallas as pl
from jax.experimental.pallas import tpu as pltpu
from jax.experimental.pallas import tpu_sc as plsc

N, E, D, ED, HID, G = 10000, 320000, 128, 16, 64, 64
NP = 10240
NC, NS, L = 2, 16, 16
NW = NC * NS
EW = E // NW
K = 80
CH = EW // K
RW = NP // NS
CW = 16
BN = 2048
NB = NP // BN
BE = 2000
F32 = jnp.float32


def _prologue_body(x_ref, nw_ref, nb_ref, wa_ref, wb_ref, h_ref, a_ref, b_ref):
    h = jnp.dot(x_ref[...], nw_ref[...], preferred_element_type=F32) + nb_ref[...]
    h_ref[...] = h
    a_ref[...] = jnp.dot(h, wa_ref[...], preferred_element_type=F32)
    b_ref[...] = jnp.dot(h, wb_ref[...], preferred_element_type=F32)


_prologue = pl.pallas_call(
    _prologue_body,
    grid=(NB,),
    in_specs=[
        pl.BlockSpec((BN, D), lambda i: (i, 0)),
        pl.BlockSpec((D, HID), lambda i: (0, 0)),
        pl.BlockSpec((1, HID), lambda i: (0, 0)),
        pl.BlockSpec((HID, HID), lambda i: (0, 0)),
        pl.BlockSpec((HID, HID), lambda i: (0, 0)),
    ],
    out_specs=[pl.BlockSpec((BN, HID), lambda i: (i, 0))] * 3,
    out_shape=[jax.ShapeDtypeStruct((NP, HID), F32)] * 3,
)


def _cpre_body(ea_ref, wc_ref, cb_ref, c_ref):
    c_ref[0] = jnp.dot(ea_ref[...], wc_ref[0], preferred_element_type=F32) + cb_ref[0]


_cpre = pl.pallas_call(
    _cpre_body,
    grid=(3, E // BE),
    in_specs=[
        pl.BlockSpec((BE, ED), lambda l, e: (e, 0)),
        pl.BlockSpec((1, ED, HID), lambda l, e: (l, 0, 0)),
        pl.BlockSpec((1, 1, HID), lambda l, e: (l, 0, 0)),
    ],
    out_specs=pl.BlockSpec((1, BE, HID), lambda l, e: (l, e, 0)),
    out_shape=jax.ShapeDtypeStruct((3, E, HID), F32),
)


def _make_edge_sc(layer, with_cnt):
    out_type = [jax.ShapeDtypeStruct((NC, NP, HID), F32)]
    if with_cnt:
        out_type.append(jax.ShapeDtypeStruct((NC, NP, CW), F32))
    scratch = [
        pltpu.VMEM((1, K), jnp.int32),
        pltpu.VMEM((1, K), jnp.int32),
        pltpu.VMEM((K, HID), F32),
        pltpu.VMEM((K, HID), F32),
        pltpu.VMEM((K, HID), F32),
        pltpu.VMEM((K, HID), F32),
        pltpu.VMEM((128, HID), F32),
        pltpu.VMEM_SHARED((NP, HID), F32),
        pltpu.SemaphoreType.DMA,
        pltpu.SemaphoreType.DMA,
        pltpu.SemaphoreType.DMA,
    ]
    if with_cnt:
        scratch += [
            pltpu.VMEM((K, CW), F32),
            pltpu.VMEM((128, CW), F32),
            pltpu.VMEM_SHARED((NP, CW), F32),
        ]

    def body(a_hbm, b_hbm, c_hbm, dst_hbm, src_hbm, s_out, *rest):
        if with_cnt:
            cnt_out, *rest = rest
        (dsti, srci, arow, brow, crow, res, zbuf, s_sh,
         sem_a, sem_b, sem_c, *cextra) = rest
        cidx = lax.axis_index("c")
        sidx = lax.axis_index("s")
        wid = sidx * NC + cidx
        base = wid * EW

        def zrow(i, _):
            for j in range(HID // L):
                zbuf[i, pl.ds(j * L, L)] = jnp.zeros((L,), F32)
            return 0
        lax.fori_loop(0, 128, zrow, 0)
        for r in range(RW // 128):
            pltpu.sync_copy(zbuf, s_sh.at[pl.ds(sidx * RW + r * 128, 128)])
        if with_cnt:
            ones, zcbuf, cnt_sh = cextra
            def crow_init(i, _):
                ones[i, pl.ds(0, L)] = jnp.ones((L,), F32)
                zcbuf[i, pl.ds(0, L)] = jnp.zeros((L,), F32)
                return 0
            lax.fori_loop(0, 128, crow_init, 0)
            for r in range(RW // 128):
                pltpu.sync_copy(zcbuf, cnt_sh.at[pl.ds(sidx * RW + r * 128, 128)])
        plsc.subcore_barrier()

        def chunk(g, _):
            eb = base + g * K
            pltpu.sync_copy(dst_hbm.at[pl.ds(eb, K)], dsti.at[0])
            pltpu.sync_copy(src_hbm.at[pl.ds(eb, K)], srci.at[0])
            ca = pltpu.async_copy(a_hbm.at[dsti.at[0]], arow, sem_a)
            cb = pltpu.async_copy(b_hbm.at[srci.at[0]], brow, sem_b)
            cc = pltpu.async_copy(c_hbm.at[layer, pl.ds(eb, K)], crow, sem_c)
            ca.wait()
            cb.wait()
            cc.wait()

            def erow(i, _):
                for j in range(HID // L):
                    sl = pl.ds(j * L, L)
                    res[i, sl] = jnp.maximum(arow[i, sl] + brow[i, sl] + crow[i, sl], 0.0)
                return 0
            lax.fori_loop(0, K, erow, 0)
            pltpu.sync_copy(res, s_sh.at[dsti.at[0]], add=True)
            if with_cnt:
                pltpu.sync_copy(cextra[0], cextra[2].at[dsti.at[0]], add=True)
            return 0
        lax.fori_loop(0, CH, chunk, 0)

        plsc.subcore_barrier()
        for r in range(RW // 128):
            rows = pl.ds(sidx * RW + r * 128, 128)
            pltpu.sync_copy(s_sh.at[rows], s_out.at[cidx, rows])
            if with_cnt:
                pltpu.sync_copy(cextra[2].at[rows], cnt_out.at[cidx, rows])

    return pl.kernel(
        body,
        out_type=out_type,
        mesh=plsc.VectorSubcoreMesh(core_axis_name="c", subcore_axis_name="s"),
        scratch_types=scratch,
        compiler_params=pltpu.CompilerParams(use_tc_tiling_on_sc=False),
    )


_edge_sc = [_make_edge_sc(0, True), _make_edge_sc(1, False), _make_edge_sc(2, False)]


def _node_body(h_ref, s_ref, cnt_ref, u1a_ref, m2u_ref, b2u_ref, u1b_ref,
               u2w_ref, u2b_ref, wa_ref, wb_ref, hn_ref, an_ref, bn_ref):
    h = h_ref[...]
    ss = s_ref[0] + s_ref[1]
    c2 = cnt_ref[0, :, :1] + cnt_ref[1, :, :1]
    t = (jnp.dot(h, u1a_ref[...], preferred_element_type=F32)
         + jnp.dot(ss, m2u_ref[...], preferred_element_type=F32)
         + c2 * b2u_ref[...] + u1b_ref[...])
    u = jnp.dot(jnp.maximum(t, 0.0), u2w_ref[...], preferred_element_type=F32) + u2b_ref[...]
    hn = h + u
    hn_ref[...] = hn
    an_ref[...] = jnp.dot(hn, wa_ref[...], preferred_element_type=F32)
    bn_ref[...] = jnp.dot(hn, wb_ref[...], preferred_element_type=F32)


_node_update = pl.pallas_call(
    _node_body,
    grid=(NB,),
    in_specs=[
        pl.BlockSpec((BN, HID), lambda i: (i, 0)),
        pl.BlockSpec((NC, BN, HID), lambda i: (0, i, 0)),
        pl.BlockSpec((NC, BN, CW), lambda i: (0, i, 0)),
    ] + [pl.BlockSpec((HID, HID), lambda i: (0, 0)),
         pl.BlockSpec((HID, HID), lambda i: (0, 0)),
         pl.BlockSpec((1, HID), lambda i: (0, 0)),
         pl.BlockSpec((1, HID), lambda i: (0, 0)),
         pl.BlockSpec((HID, HID), lambda i: (0, 0)),
         pl.BlockSpec((1, HID), lambda i: (0, 0)),
         pl.BlockSpec((HID, HID), lambda i: (0, 0)),
         pl.BlockSpec((HID, HID), lambda i: (0, 0))],
    out_specs=[pl.BlockSpec((BN, HID), lambda i: (i, 0))] * 3,
    out_shape=[jax.ShapeDtypeStruct((NP, HID), F32)] * 3,
)


def _final_body(h_ref, s_ref, cnt_ref, batch_ref, gf_ref,
                u1a_ref, m2u_ref, b2u_ref, u1b_ref, u2w_ref, u2b_ref,
                r1a_ref, gw_ref, rb1_ref, r2w_ref, r2b_ref, r3w_ref, r3b_ref,
                out_ref, psum, pcnt):
    i = pl.program_id(0)

    @pl.when(i == 0)
    def _():
        psum[...] = jnp.zeros_like(psum)
        pcnt[...] = jnp.zeros_like(pcnt)

    h = h_ref[...]
    ss = s_ref[0] + s_ref[1]
    c2 = cnt_ref[0, :, :1] + cnt_ref[1, :, :1]
    t = (jnp.dot(h, u1a_ref[...], preferred_element_type=F32)
         + jnp.dot(ss, m2u_ref[...], preferred_element_type=F32)
         + c2 * b2u_ref[...] + u1b_ref[...])
    u = jnp.dot(jnp.maximum(t, 0.0), u2w_ref[...], preferred_element_type=F32) + u2b_ref[...]
    hn = h + u
    oh = (batch_ref[...] == lax.broadcasted_iota(jnp.int32, (1, G), 1)).astype(F32)
    dn = (((0,), (0,)), ((), ()))
    psum[...] += lax.dot_general(oh, hn, dn, preferred_element_type=F32)
    pcnt[...] += lax.dot_general(oh, jnp.ones((BN, 1), F32), dn, preferred_element_type=F32)

    @pl.when(i == NB - 1)
    def _():
        pooled = psum[...] / jnp.maximum(pcnt[...], 1.0)
        t1 = (jnp.dot(pooled, r1a_ref[...], preferred_element_type=F32)
              + jnp.dot(gf_ref[...], gw_ref[...], preferred_element_type=F32)
              + rb1_ref[...])
        r1 = jnp.maximum(t1, 0.0)
        r2 = jnp.maximum(jnp.dot(r1, r2w_ref[...], preferred_element_type=F32) + r2b_ref[...], 0.0)
        out_ref[...] = jnp.dot(r2, r3w_ref[...], preferred_element_type=F32) + r3b_ref[...]


_final = pl.pallas_call(
    _final_body,
    grid=(NB,),
    in_specs=[
        pl.BlockSpec((BN, HID), lambda i: (i, 0)),
        pl.BlockSpec((NC, BN, HID), lambda i: (0, i, 0)),
        pl.BlockSpec((NC, BN, CW), lambda i: (0, i, 0)),
        pl.BlockSpec((BN, 1), lambda i: (i, 0)),
        pl.BlockSpec((G, 1), lambda i: (0, 0)),
    ] + [pl.BlockSpec((HID, HID), lambda i: (0, 0)),
         pl.BlockSpec((HID, HID), lambda i: (0, 0)),
         pl.BlockSpec((1, HID), lambda i: (0, 0)),
         pl.BlockSpec((1, HID), lambda i: (0, 0)),
         pl.BlockSpec((HID, HID), lambda i: (0, 0)),
         pl.BlockSpec((1, HID), lambda i: (0, 0)),
         pl.BlockSpec((HID, HID), lambda i: (0, 0)),
         pl.BlockSpec((1, HID), lambda i: (0, 0)),
         pl.BlockSpec((1, HID), lambda i: (0, 0)),
         pl.BlockSpec((HID, HID // 2), lambda i: (0, 0)),
         pl.BlockSpec((1, HID // 2), lambda i: (0, 0)),
         pl.BlockSpec((HID // 2, 1), lambda i: (0, 0)),
         pl.BlockSpec((1, 1), lambda i: (0, 0))],
    out_specs=pl.BlockSpec((G, 1), lambda i: (0, 0)),
    out_shape=jax.ShapeDtypeStruct((G, 1), F32),
    scratch_shapes=[pltpu.VMEM((G, HID), F32), pltpu.VMEM((G, 1), F32)],
)


def kernel(x, edge_index, edge_attr, batch, global_feature, params):
    p = params
    layers = p['layers']

    xp = jnp.zeros((NP, D), F32).at[:N].set(x)
    batch_p = jnp.full((NP, 1), G, jnp.int32).at[:N, 0].set(batch.astype(jnp.int32))
    src = edge_index[0].astype(jnp.int32)
    dst = edge_index[1].astype(jnp.int32)

    W1a = [lp['m1_w'][:HID] for lp in layers]
    W1b = [lp['m1_w'][HID:2 * HID] for lp in layers]
    wc_all = jnp.stack([p['edge_w'] @ lp['m1_w'][2 * HID:] for lp in layers])
    cb_all = jnp.stack([(p['edge_b'] @ lp['m1_w'][2 * HID:] + lp['m1_b'])[None]
                        for lp in layers])
    U1a = [lp['u1_w'][:HID] for lp in layers]
    M2U = [lp['m2_w'] @ lp['u1_w'][HID:] for lp in layers]
    b2u = [(lp['m2_b'] @ lp['u1_w'][HID:])[None] for lp in layers]
    u1b = [lp['u1_b'][None] for lp in layers]
    R1b = p['r1_w'][HID:]
    gw = p['glob_w'] @ R1b
    rb1 = (p['r1_b'] + p['glob_b'] @ R1b)[None]

    h, A, B = _prologue(xp, p['node_w'], p['node_b'][None], W1a[0], W1b[0])
    C = _cpre(edge_attr, wc_all, cb_all)

    S, cnt = _edge_sc[0](A, B, C, dst, src)
    for l in range(2):
        h, A, B = _node_update(h, S, cnt, U1a[l], M2U[l], b2u[l], u1b[l],
                               layers[l]['u2_w'], layers[l]['u2_b'][None],
                               W1a[l + 1], W1b[l + 1])
        (S,) = _edge_sc[l + 1](A, B, C, dst, src)

    return _final(h, S, cnt, batch_p, global_feature,
                  U1a[2], M2U[2], b2u[2], u1b[2],
                  layers[2]['u2_w'], layers[2]['u2_b'][None],
                  p['r1_w'][:HID], gw, rb1,
                  p['r2_w'], p['r2_b'][None], p['r3_w'], p['r3_b'][None])


# speedup vs baseline: 2.9403x; 1.0339x over previous
import functools

import jax
import jax.numpy as jnp
from jax import lax
from jax.experimental import pallas as pl
from jax.experimental.pallas import tpu as pltpu
from jax.experimental.pallas import tpu_sc as plsc

N, E, D, ED, HID, G = 10000, 320000, 128, 16, 64, 64
NP = 10240
NC, NS, L = 2, 16, 16
NW = NC * NS
EW = E // NW
K = 80
CH = EW // K
RW = NP // NS
CW = 16
BN = 2048
NB = NP // BN
BE = 2000
F32 = jnp.float32


def _prologue_body(x_ref, nw_ref, nb_ref, wa_ref, wb_ref, h_ref, a_ref, b_ref):
    h = jnp.dot(x_ref[...], nw_ref[...], preferred_element_type=F32) + nb_ref[...]
    h_ref[...] = h
    a_ref[...] = jnp.dot(h, wa_ref[...], preferred_element_type=F32)
    b_ref[...] = jnp.dot(h, wb_ref[...], preferred_element_type=F32)


_prologue = pl.pallas_call(
    _prologue_body,
    grid=(NB,),
    in_specs=[
        pl.BlockSpec((BN, D), lambda i: (i, 0)),
        pl.BlockSpec((D, HID), lambda i: (0, 0)),
        pl.BlockSpec((1, HID), lambda i: (0, 0)),
        pl.BlockSpec((HID, HID), lambda i: (0, 0)),
        pl.BlockSpec((HID, HID), lambda i: (0, 0)),
    ],
    out_specs=[pl.BlockSpec((BN, HID), lambda i: (i, 0))] * 3,
    out_shape=[jax.ShapeDtypeStruct((NP, HID), F32)] * 3,
)


CR = E * HID // 128
BTH = 6400


def _cpre_body(ev_ref, od_ref, wc_ref, cb_ref, c_ref):
    dn = (((0,), (0,)), ((), ()))
    t_ev = lax.dot_general(ev_ref[...], wc_ref[0], dn,
                           preferred_element_type=F32) + cb_ref[0]
    t_od = lax.dot_general(od_ref[...], wc_ref[0], dn,
                           preferred_element_type=F32) + cb_ref[0]
    c_ref[0] = jnp.concatenate([t_ev, t_od], axis=1)


_cpre = pl.pallas_call(
    _cpre_body,
    grid=(3, CR // BTH),
    in_specs=[
        pl.BlockSpec((ED, BTH), lambda l, e: (0, e)),
        pl.BlockSpec((ED, BTH), lambda l, e: (0, e)),
        pl.BlockSpec((1, ED, HID), lambda l, e: (l, 0, 0)),
        pl.BlockSpec((1, 1, HID), lambda l, e: (l, 0, 0)),
    ],
    out_specs=pl.BlockSpec((1, BTH, 128), lambda l, e: (l, e, 0)),
    out_shape=jax.ShapeDtypeStruct((3, CR, 128), F32),
)


def _make_edge_sc(layer, with_cnt):
    out_type = [jax.ShapeDtypeStruct((NC, NP, HID), F32)]
    if with_cnt:
        out_type.append(jax.ShapeDtypeStruct((NC, NP, CW), F32))
    scratch = [
        pltpu.VMEM((1, K), jnp.int32),
        pltpu.VMEM((1, K), jnp.int32),
        pltpu.VMEM((K, HID), F32),
        pltpu.VMEM((K, HID), F32),
        pltpu.VMEM((K // 2, 128), F32),
        pltpu.VMEM((K, HID), F32),
        pltpu.VMEM((128, HID), F32),
        pltpu.VMEM_SHARED((NP, HID), F32),
        pltpu.SemaphoreType.DMA,
        pltpu.SemaphoreType.DMA,
        pltpu.SemaphoreType.DMA,
    ]
    if with_cnt:
        scratch += [
            pltpu.VMEM((K, CW), F32),
            pltpu.VMEM((128, CW), F32),
            pltpu.VMEM_SHARED((NP, CW), F32),
        ]

    def body(a_hbm, b_hbm, c_hbm, dst_hbm, src_hbm, s_out, *rest):
        if with_cnt:
            cnt_out, *rest = rest
        (dsti, srci, arow, brow, crow, res, zbuf, s_sh,
         sem_a, sem_b, sem_c, *cextra) = rest
        cidx = lax.axis_index("c")
        sidx = lax.axis_index("s")
        wid = sidx * NC + cidx
        base = wid * EW

        def zrow(i, _):
            for j in range(HID // L):
                zbuf[i, pl.ds(j * L, L)] = jnp.zeros((L,), F32)
            return 0
        lax.fori_loop(0, 128, zrow, 0)
        for r in range(RW // 128):
            pltpu.sync_copy(zbuf, s_sh.at[pl.ds(sidx * RW + r * 128, 128)])
        if with_cnt:
            ones, zcbuf, cnt_sh = cextra
            def crow_init(i, _):
                ones[i, pl.ds(0, L)] = jnp.ones((L,), F32)
                zcbuf[i, pl.ds(0, L)] = jnp.zeros((L,), F32)
                return 0
            lax.fori_loop(0, 128, crow_init, 0)
            for r in range(RW // 128):
                pltpu.sync_copy(zcbuf, cnt_sh.at[pl.ds(sidx * RW + r * 128, 128)])
        plsc.subcore_barrier()

        def chunk(g, _):
            eb = base + g * K
            pltpu.sync_copy(dst_hbm.at[pl.ds(eb, K)], dsti.at[0])
            pltpu.sync_copy(src_hbm.at[pl.ds(eb, K)], srci.at[0])
            ca = pltpu.async_copy(a_hbm.at[dsti.at[0]], arow, sem_a)
            cb = pltpu.async_copy(b_hbm.at[srci.at[0]], brow, sem_b)
            cc = pltpu.async_copy(
                c_hbm.at[layer, pl.ds(base // 2 + g * (K // 2), K // 2)], crow, sem_c)
            ca.wait()
            cb.wait()
            cc.wait()

            def erow(ip, _):
                for le in range(2):
                    for j in range(HID // L):
                        sl = pl.ds(j * L, L)
                        e = 2 * ip + le
                        res[e, sl] = jnp.maximum(
                            arow[e, sl] + brow[e, sl]
                            + crow[ip, pl.ds(le * HID + j * L, L)], 0.0)
                return 0
            lax.fori_loop(0, K // 2, erow, 0)
            pltpu.sync_copy(res, s_sh.at[dsti.at[0]], add=True)
            if with_cnt:
                pltpu.sync_copy(cextra[0], cextra[2].at[dsti.at[0]], add=True)
            return 0
        lax.fori_loop(0, CH, chunk, 0)

        plsc.subcore_barrier()
        for r in range(RW // 128):
            rows = pl.ds(sidx * RW + r * 128, 128)
            pltpu.sync_copy(s_sh.at[rows], s_out.at[cidx, rows])
            if with_cnt:
                pltpu.sync_copy(cextra[2].at[rows], cnt_out.at[cidx, rows])

    return pl.kernel(
        body,
        out_type=out_type,
        mesh=plsc.VectorSubcoreMesh(core_axis_name="c", subcore_axis_name="s"),
        scratch_types=scratch,
        compiler_params=pltpu.CompilerParams(use_tc_tiling_on_sc=False),
    )


_edge_sc = [_make_edge_sc(0, True), _make_edge_sc(1, False), _make_edge_sc(2, False)]


def _node_body(h_ref, s_ref, cnt_ref, u1a_ref, m2u_ref, b2u_ref, u1b_ref,
               u2w_ref, u2b_ref, wa_ref, wb_ref, hn_ref, an_ref, bn_ref):
    h = h_ref[...]
    ss = s_ref[0] + s_ref[1]
    c2 = cnt_ref[0, :, :1] + cnt_ref[1, :, :1]
    t = (jnp.dot(h, u1a_ref[...], preferred_element_type=F32)
         + jnp.dot(ss, m2u_ref[...], preferred_element_type=F32)
         + c2 * b2u_ref[...] + u1b_ref[...])
    u = jnp.dot(jnp.maximum(t, 0.0), u2w_ref[...], preferred_element_type=F32) + u2b_ref[...]
    hn = h + u
    hn_ref[...] = hn
    an_ref[...] = jnp.dot(hn, wa_ref[...], preferred_element_type=F32)
    bn_ref[...] = jnp.dot(hn, wb_ref[...], preferred_element_type=F32)


_node_update = pl.pallas_call(
    _node_body,
    grid=(NB,),
    in_specs=[
        pl.BlockSpec((BN, HID), lambda i: (i, 0)),
        pl.BlockSpec((NC, BN, HID), lambda i: (0, i, 0)),
        pl.BlockSpec((NC, BN, CW), lambda i: (0, i, 0)),
    ] + [pl.BlockSpec((HID, HID), lambda i: (0, 0)),
         pl.BlockSpec((HID, HID), lambda i: (0, 0)),
         pl.BlockSpec((1, HID), lambda i: (0, 0)),
         pl.BlockSpec((1, HID), lambda i: (0, 0)),
         pl.BlockSpec((HID, HID), lambda i: (0, 0)),
         pl.BlockSpec((1, HID), lambda i: (0, 0)),
         pl.BlockSpec((HID, HID), lambda i: (0, 0)),
         pl.BlockSpec((HID, HID), lambda i: (0, 0))],
    out_specs=[pl.BlockSpec((BN, HID), lambda i: (i, 0))] * 3,
    out_shape=[jax.ShapeDtypeStruct((NP, HID), F32)] * 3,
)


def _final_body(h_ref, s_ref, cnt_ref, batch_ref, gf_ref,
                u1a_ref, m2u_ref, b2u_ref, u1b_ref, u2w_ref, u2b_ref,
                r1a_ref, gw_ref, rb1_ref, r2w_ref, r2b_ref, r3w_ref, r3b_ref,
                out_ref, psum, pcnt):
    i = pl.program_id(0)

    @pl.when(i == 0)
    def _():
        psum[...] = jnp.zeros_like(psum)
        pcnt[...] = jnp.zeros_like(pcnt)

    h = h_ref[...]
    ss = s_ref[0] + s_ref[1]
    c2 = cnt_ref[0, :, :1] + cnt_ref[1, :, :1]
    t = (jnp.dot(h, u1a_ref[...], preferred_element_type=F32)
         + jnp.dot(ss, m2u_ref[...], preferred_element_type=F32)
         + c2 * b2u_ref[...] + u1b_ref[...])
    u = jnp.dot(jnp.maximum(t, 0.0), u2w_ref[...], preferred_element_type=F32) + u2b_ref[...]
    hn = h + u
    oh = (batch_ref[...] == lax.broadcasted_iota(jnp.int32, (1, G), 1)).astype(F32)
    dn = (((0,), (0,)), ((), ()))
    psum[...] += lax.dot_general(oh, hn, dn, preferred_element_type=F32)
    pcnt[...] += lax.dot_general(oh, jnp.ones((BN, 1), F32), dn, preferred_element_type=F32)

    @pl.when(i == NB - 1)
    def _():
        pooled = psum[...] / jnp.maximum(pcnt[...], 1.0)
        t1 = (jnp.dot(pooled, r1a_ref[...], preferred_element_type=F32)
              + jnp.dot(gf_ref[...], gw_ref[...], preferred_element_type=F32)
              + rb1_ref[...])
        r1 = jnp.maximum(t1, 0.0)
        r2 = jnp.maximum(jnp.dot(r1, r2w_ref[...], preferred_element_type=F32) + r2b_ref[...], 0.0)
        out_ref[...] = jnp.dot(r2, r3w_ref[...], preferred_element_type=F32) + r3b_ref[...]


_final = pl.pallas_call(
    _final_body,
    grid=(NB,),
    in_specs=[
        pl.BlockSpec((BN, HID), lambda i: (i, 0)),
        pl.BlockSpec((NC, BN, HID), lambda i: (0, i, 0)),
        pl.BlockSpec((NC, BN, CW), lambda i: (0, i, 0)),
        pl.BlockSpec((BN, 1), lambda i: (i, 0)),
        pl.BlockSpec((G, 1), lambda i: (0, 0)),
    ] + [pl.BlockSpec((HID, HID), lambda i: (0, 0)),
         pl.BlockSpec((HID, HID), lambda i: (0, 0)),
         pl.BlockSpec((1, HID), lambda i: (0, 0)),
         pl.BlockSpec((1, HID), lambda i: (0, 0)),
         pl.BlockSpec((HID, HID), lambda i: (0, 0)),
         pl.BlockSpec((1, HID), lambda i: (0, 0)),
         pl.BlockSpec((HID, HID), lambda i: (0, 0)),
         pl.BlockSpec((1, HID), lambda i: (0, 0)),
         pl.BlockSpec((1, HID), lambda i: (0, 0)),
         pl.BlockSpec((HID, HID // 2), lambda i: (0, 0)),
         pl.BlockSpec((1, HID // 2), lambda i: (0, 0)),
         pl.BlockSpec((HID // 2, 1), lambda i: (0, 0)),
         pl.BlockSpec((1, 1), lambda i: (0, 0))],
    out_specs=pl.BlockSpec((G, 1), lambda i: (0, 0)),
    out_shape=jax.ShapeDtypeStruct((G, 1), F32),
    scratch_shapes=[pltpu.VMEM((G, HID), F32), pltpu.VMEM((G, 1), F32)],
)


def kernel(x, edge_index, edge_attr, batch, global_feature, params):
    p = params
    layers = p['layers']

    xp = jnp.zeros((NP, D), F32).at[:N].set(x)
    batch_p = jnp.full((NP, 1), G, jnp.int32).at[:N, 0].set(batch.astype(jnp.int32))
    src = edge_index[0].astype(jnp.int32)
    dst = edge_index[1].astype(jnp.int32)

    W1a = [lp['m1_w'][:HID] for lp in layers]
    W1b = [lp['m1_w'][HID:2 * HID] for lp in layers]
    wc_all = jnp.stack([p['edge_w'] @ lp['m1_w'][2 * HID:] for lp in layers])
    cb_all = jnp.stack([(p['edge_b'] @ lp['m1_w'][2 * HID:] + lp['m1_b'])[None]
                        for lp in layers])
    ea_t = edge_attr.T
    ea_ev = ea_t[:, 0::2]
    ea_od = ea_t[:, 1::2]
    U1a = [lp['u1_w'][:HID] for lp in layers]
    M2U = [lp['m2_w'] @ lp['u1_w'][HID:] for lp in layers]
    b2u = [(lp['m2_b'] @ lp['u1_w'][HID:])[None] for lp in layers]
    u1b = [lp['u1_b'][None] for lp in layers]
    R1b = p['r1_w'][HID:]
    gw = p['glob_w'] @ R1b
    rb1 = (p['r1_b'] + p['glob_b'] @ R1b)[None]

    h, A, B = _prologue(xp, p['node_w'], p['node_b'][None], W1a[0], W1b[0])
    C = _cpre(ea_ev, ea_od, wc_all, cb_all)

    S, cnt = _edge_sc[0](A, B, C, dst, src)
    for l in range(2):
        h, A, B = _node_update(h, S, cnt, U1a[l], M2U[l], b2u[l], u1b[l],
                               layers[l]['u2_w'], layers[l]['u2_b'][None],
                               W1a[l + 1], W1b[l + 1])
        (S,) = _edge_sc[l + 1](A, B, C, dst, src)

    return _final(h, S, cnt, batch_p, global_feature,
                  U1a[2], M2U[2], b2u[2], u1b[2],
                  layers[2]['u2_w'], layers[2]['u2_b'][None],
                  p['r1_w'][:HID], gw, rb1,
                  p['r2_w'], p['r2_b'][None], p['r3_w'], p['r3_b'][None])


# speedup vs baseline: 4.3556x; 1.4814x over previous
import functools

import jax
import jax.numpy as jnp
from jax import lax
from jax.experimental import pallas as pl
from jax.experimental.pallas import tpu as pltpu
from jax.experimental.pallas import tpu_sc as plsc

N, E, D, ED, HID, G = 10000, 320000, 128, 16, 64, 64
NP = 10240
NC, NS, L = 2, 16, 16
NW = NC * NS
EW = E // NW
K = 80
CH = EW // K
RW = NP // NS
CW = 16
BN = 2048
NB = NP // BN
BE = 2000
F32 = jnp.float32


def _prologue_body(x_ref, nw_ref, nb_ref, wa_ref, wb_ref, h_ref, a_ref, b_ref):
    h = jnp.dot(x_ref[...], nw_ref[...], preferred_element_type=F32) + nb_ref[...]
    h_ref[...] = h
    a_ref[...] = jnp.dot(h, wa_ref[...], preferred_element_type=F32)
    b_ref[...] = jnp.dot(h, wb_ref[...], preferred_element_type=F32)


_prologue = pl.pallas_call(
    _prologue_body,
    grid=(NB,),
    in_specs=[
        pl.BlockSpec((BN, D), lambda i: (i, 0)),
        pl.BlockSpec((D, HID), lambda i: (0, 0)),
        pl.BlockSpec((1, HID), lambda i: (0, 0)),
        pl.BlockSpec((HID, HID), lambda i: (0, 0)),
        pl.BlockSpec((HID, HID), lambda i: (0, 0)),
    ],
    out_specs=[pl.BlockSpec((BN, HID), lambda i: (i, 0))] * 3,
    out_shape=[jax.ShapeDtypeStruct((NP, HID), F32)] * 3,
)


CR = E // 2
BTH = 6400
NBH = CR // BTH


def _cpre_body(lh_ref, rh_ref, wc_ref, cb_ref, c_ref):
    dn = (((0,), (0,)), ((), ()))
    t_lh = lax.dot_general(lh_ref[...], wc_ref[0], dn,
                           preferred_element_type=F32) + cb_ref[0]
    t_rh = lax.dot_general(rh_ref[...], wc_ref[0], dn,
                           preferred_element_type=F32) + cb_ref[0]
    c_ref[0] = jnp.concatenate([t_lh, t_rh], axis=1)


_cpre = pl.pallas_call(
    _cpre_body,
    grid=(3, NBH),
    in_specs=[
        pl.BlockSpec((ED, BTH), lambda l, e: (0, e)),
        pl.BlockSpec((ED, BTH), lambda l, e: (0, e + NBH)),
        pl.BlockSpec((1, ED, HID), lambda l, e: (l, 0, 0)),
        pl.BlockSpec((1, 1, HID), lambda l, e: (l, 0, 0)),
    ],
    out_specs=pl.BlockSpec((1, BTH, 128), lambda l, e: (l, e, 0)),
    out_shape=jax.ShapeDtypeStruct((3, CR, 128), F32),
)


def _make_edge_sc(layer, with_cnt):
    out_type = [jax.ShapeDtypeStruct((NC, NP, HID), F32)]
    if with_cnt:
        out_type.append(jax.ShapeDtypeStruct((NC, NP, CW), F32))
    scratch = [
        pltpu.VMEM((1, K), jnp.int32),
        pltpu.VMEM((1, K), jnp.int32),
        pltpu.VMEM((K, HID), F32),
        pltpu.VMEM((K, HID), F32),
        pltpu.VMEM((K, HID), F32),
        pltpu.VMEM((K, HID), F32),
        pltpu.VMEM((128, HID), F32),
        pltpu.VMEM_SHARED((NP, HID), F32),
        pltpu.SemaphoreType.DMA,
        pltpu.SemaphoreType.DMA,
        pltpu.SemaphoreType.DMA,
    ]
    if with_cnt:
        scratch += [
            pltpu.VMEM((K, CW), F32),
            pltpu.VMEM((128, CW), F32),
            pltpu.VMEM_SHARED((NP, CW), F32),
        ]

    def body(a_hbm, b_hbm, c_hbm, dst_hbm, src_hbm, s_out, *rest):
        if with_cnt:
            cnt_out, *rest = rest
        (dsti, srci, arow, brow, crow, res, zbuf, s_sh,
         sem_a, sem_b, sem_c, *cextra) = rest
        cidx = lax.axis_index("c")
        sidx = lax.axis_index("s")
        wid = sidx * NC + cidx
        base = wid * EW
        in_lh = wid < NW // 2
        rbase = jnp.where(in_lh, wid, wid - NW // 2) * EW

        def zrow(i, _):
            for j in range(HID // L):
                zbuf[i, pl.ds(j * L, L)] = jnp.zeros((L,), F32)
            return 0
        lax.fori_loop(0, 128, zrow, 0)
        for r in range(RW // 128):
            pltpu.sync_copy(zbuf, s_sh.at[pl.ds(sidx * RW + r * 128, 128)])
        if with_cnt:
            ones, zcbuf, cnt_sh = cextra
            def crow_init(i, _):
                ones[i, pl.ds(0, L)] = jnp.ones((L,), F32)
                zcbuf[i, pl.ds(0, L)] = jnp.zeros((L,), F32)
                return 0
            lax.fori_loop(0, 128, crow_init, 0)
            for r in range(RW // 128):
                pltpu.sync_copy(zcbuf, cnt_sh.at[pl.ds(sidx * RW + r * 128, 128)])
        plsc.subcore_barrier()

        def chunk(g, _):
            eb = base + g * K
            pltpu.sync_copy(dst_hbm.at[pl.ds(eb, K)], dsti.at[0])
            pltpu.sync_copy(src_hbm.at[pl.ds(eb, K)], srci.at[0])
            ca = pltpu.async_copy(a_hbm.at[dsti.at[0]], arow, sem_a)
            cb = pltpu.async_copy(b_hbm.at[srci.at[0]], brow, sem_b)
            crows = pl.ds(rbase + g * K, K)

            @pl.when(in_lh)
            def _():
                pltpu.sync_copy(c_hbm.at[layer, crows, pl.ds(0, HID)], crow)

            @pl.when(jnp.logical_not(in_lh))
            def _():
                pltpu.sync_copy(c_hbm.at[layer, crows, pl.ds(HID, HID)], crow)

            ca.wait()
            cb.wait()

            def erow(i, _):
                for j in range(HID // L):
                    sl = pl.ds(j * L, L)
                    res[i, sl] = jnp.maximum(
                        arow[i, sl] + brow[i, sl] + crow[i, sl], 0.0)
                return 0
            lax.fori_loop(0, K, erow, 0)
            pltpu.sync_copy(res, s_sh.at[dsti.at[0]], add=True)
            if with_cnt:
                pltpu.sync_copy(cextra[0], cextra[2].at[dsti.at[0]], add=True)
            return 0
        lax.fori_loop(0, CH, chunk, 0)

        plsc.subcore_barrier()
        for r in range(RW // 128):
            rows = pl.ds(sidx * RW + r * 128, 128)
            pltpu.sync_copy(s_sh.at[rows], s_out.at[cidx, rows])
            if with_cnt:
                pltpu.sync_copy(cextra[2].at[rows], cnt_out.at[cidx, rows])

    return pl.kernel(
        body,
        out_type=out_type,
        mesh=plsc.VectorSubcoreMesh(core_axis_name="c", subcore_axis_name="s"),
        scratch_types=scratch,
        compiler_params=pltpu.CompilerParams(use_tc_tiling_on_sc=False),
    )


_edge_sc = [_make_edge_sc(0, True), _make_edge_sc(1, False), _make_edge_sc(2, False)]


def _node_body(h_ref, s_ref, cnt_ref, u1a_ref, m2u_ref, b2u_ref, u1b_ref,
               u2w_ref, u2b_ref, wa_ref, wb_ref, hn_ref, an_ref, bn_ref):
    h = h_ref[...]
    ss = s_ref[0] + s_ref[1]
    c2 = cnt_ref[0, :, :1] + cnt_ref[1, :, :1]
    t = (jnp.dot(h, u1a_ref[...], preferred_element_type=F32)
         + jnp.dot(ss, m2u_ref[...], preferred_element_type=F32)
         + c2 * b2u_ref[...] + u1b_ref[...])
    u = jnp.dot(jnp.maximum(t, 0.0), u2w_ref[...], preferred_element_type=F32) + u2b_ref[...]
    hn = h + u
    hn_ref[...] = hn
    an_ref[...] = jnp.dot(hn, wa_ref[...], preferred_element_type=F32)
    bn_ref[...] = jnp.dot(hn, wb_ref[...], preferred_element_type=F32)


_node_update = pl.pallas_call(
    _node_body,
    grid=(NB,),
    in_specs=[
        pl.BlockSpec((BN, HID), lambda i: (i, 0)),
        pl.BlockSpec((NC, BN, HID), lambda i: (0, i, 0)),
        pl.BlockSpec((NC, BN, CW), lambda i: (0, i, 0)),
    ] + [pl.BlockSpec((HID, HID), lambda i: (0, 0)),
         pl.BlockSpec((HID, HID), lambda i: (0, 0)),
         pl.BlockSpec((1, HID), lambda i: (0, 0)),
         pl.BlockSpec((1, HID), lambda i: (0, 0)),
         pl.BlockSpec((HID, HID), lambda i: (0, 0)),
         pl.BlockSpec((1, HID), lambda i: (0, 0)),
         pl.BlockSpec((HID, HID), lambda i: (0, 0)),
         pl.BlockSpec((HID, HID), lambda i: (0, 0))],
    out_specs=[pl.BlockSpec((BN, HID), lambda i: (i, 0))] * 3,
    out_shape=[jax.ShapeDtypeStruct((NP, HID), F32)] * 3,
)


def _final_body(h_ref, s_ref, cnt_ref, batch_ref, gf_ref,
                u1a_ref, m2u_ref, b2u_ref, u1b_ref, u2w_ref, u2b_ref,
                r1a_ref, gw_ref, rb1_ref, r2w_ref, r2b_ref, r3w_ref, r3b_ref,
                out_ref, psum, pcnt):
    i = pl.program_id(0)

    @pl.when(i == 0)
    def _():
        psum[...] = jnp.zeros_like(psum)
        pcnt[...] = jnp.zeros_like(pcnt)

    h = h_ref[...]
    ss = s_ref[0] + s_ref[1]
    c2 = cnt_ref[0, :, :1] + cnt_ref[1, :, :1]
    t = (jnp.dot(h, u1a_ref[...], preferred_element_type=F32)
         + jnp.dot(ss, m2u_ref[...], preferred_element_type=F32)
         + c2 * b2u_ref[...] + u1b_ref[...])
    u = jnp.dot(jnp.maximum(t, 0.0), u2w_ref[...], preferred_element_type=F32) + u2b_ref[...]
    hn = h + u
    oh = (batch_ref[...] == lax.broadcasted_iota(jnp.int32, (1, G), 1)).astype(F32)
    dn = (((0,), (0,)), ((), ()))
    psum[...] += lax.dot_general(oh, hn, dn, preferred_element_type=F32)
    pcnt[...] += lax.dot_general(oh, jnp.ones((BN, 1), F32), dn, preferred_element_type=F32)

    @pl.when(i == NB - 1)
    def _():
        pooled = psum[...] / jnp.maximum(pcnt[...], 1.0)
        t1 = (jnp.dot(pooled, r1a_ref[...], preferred_element_type=F32)
              + jnp.dot(gf_ref[...], gw_ref[...], preferred_element_type=F32)
              + rb1_ref[...])
        r1 = jnp.maximum(t1, 0.0)
        r2 = jnp.maximum(jnp.dot(r1, r2w_ref[...], preferred_element_type=F32) + r2b_ref[...], 0.0)
        out_ref[...] = jnp.dot(r2, r3w_ref[...], preferred_element_type=F32) + r3b_ref[...]


_final = pl.pallas_call(
    _final_body,
    grid=(NB,),
    in_specs=[
        pl.BlockSpec((BN, HID), lambda i: (i, 0)),
        pl.BlockSpec((NC, BN, HID), lambda i: (0, i, 0)),
        pl.BlockSpec((NC, BN, CW), lambda i: (0, i, 0)),
        pl.BlockSpec((BN, 1), lambda i: (i, 0)),
        pl.BlockSpec((G, 1), lambda i: (0, 0)),
    ] + [pl.BlockSpec((HID, HID), lambda i: (0, 0)),
         pl.BlockSpec((HID, HID), lambda i: (0, 0)),
         pl.BlockSpec((1, HID), lambda i: (0, 0)),
         pl.BlockSpec((1, HID), lambda i: (0, 0)),
         pl.BlockSpec((HID, HID), lambda i: (0, 0)),
         pl.BlockSpec((1, HID), lambda i: (0, 0)),
         pl.BlockSpec((HID, HID), lambda i: (0, 0)),
         pl.BlockSpec((1, HID), lambda i: (0, 0)),
         pl.BlockSpec((1, HID), lambda i: (0, 0)),
         pl.BlockSpec((HID, HID // 2), lambda i: (0, 0)),
         pl.BlockSpec((1, HID // 2), lambda i: (0, 0)),
         pl.BlockSpec((HID // 2, 1), lambda i: (0, 0)),
         pl.BlockSpec((1, 1), lambda i: (0, 0))],
    out_specs=pl.BlockSpec((G, 1), lambda i: (0, 0)),
    out_shape=jax.ShapeDtypeStruct((G, 1), F32),
    scratch_shapes=[pltpu.VMEM((G, HID), F32), pltpu.VMEM((G, 1), F32)],
)


def kernel(x, edge_index, edge_attr, batch, global_feature, params):
    p = params
    layers = p['layers']

    xp = jnp.zeros((NP, D), F32).at[:N].set(x)
    batch_p = jnp.full((NP, 1), G, jnp.int32).at[:N, 0].set(batch.astype(jnp.int32))
    src = edge_index[0].astype(jnp.int32)
    dst = edge_index[1].astype(jnp.int32)

    W1a = [lp['m1_w'][:HID] for lp in layers]
    W1b = [lp['m1_w'][HID:2 * HID] for lp in layers]
    wc_all = jnp.stack([p['edge_w'] @ lp['m1_w'][2 * HID:] for lp in layers])
    cb_all = jnp.stack([(p['edge_b'] @ lp['m1_w'][2 * HID:] + lp['m1_b'])[None]
                        for lp in layers])
    ea_t = edge_attr.T
    U1a = [lp['u1_w'][:HID] for lp in layers]
    M2U = [lp['m2_w'] @ lp['u1_w'][HID:] for lp in layers]
    b2u = [(lp['m2_b'] @ lp['u1_w'][HID:])[None] for lp in layers]
    u1b = [lp['u1_b'][None] for lp in layers]
    R1b = p['r1_w'][HID:]
    gw = p['glob_w'] @ R1b
    rb1 = (p['r1_b'] + p['glob_b'] @ R1b)[None]

    h, A, B = _prologue(xp, p['node_w'], p['node_b'][None], W1a[0], W1b[0])
    C = _cpre(ea_t, ea_t, wc_all, cb_all)

    S, cnt = _edge_sc[0](A, B, C, dst, src)
    for l in range(2):
        h, A, B = _node_update(h, S, cnt, U1a[l], M2U[l], b2u[l], u1b[l],
                               layers[l]['u2_w'], layers[l]['u2_b'][None],
                               W1a[l + 1], W1b[l + 1])
        (S,) = _edge_sc[l + 1](A, B, C, dst, src)

    return _final(h, S, cnt, batch_p, global_feature,
                  U1a[2], M2U[2], b2u[2], u1b[2],
                  layers[2]['u2_w'], layers[2]['u2_b'][None],
                  p['r1_w'][:HID], gw, rb1,
                  p['r2_w'], p['r2_b'][None], p['r3_w'], p['r3_b'][None])


# speedup vs baseline: 6.0804x; 1.3960x over previous
import functools

import jax
import jax.numpy as jnp
from jax import lax
from jax.experimental import pallas as pl
from jax.experimental.pallas import tpu as pltpu
from jax.experimental.pallas import tpu_sc as plsc

N, E, D, ED, HID, G = 10000, 320000, 128, 16, 64, 64
NP = 10240
NC, NS, L = 2, 16, 16
NW = NC * NS
EW = E // NW
K = 80
CH = EW // K
RW = NP // NS
CW = 16
BN = 2048
NB = NP // BN
BE = 2000
F32 = jnp.float32


def _prologue_body(x_ref, nw_ref, nb_ref, wa_ref, wb_ref, h_ref, a_ref, b_ref):
    h = jnp.dot(x_ref[...], nw_ref[...], preferred_element_type=F32) + nb_ref[...]
    h_ref[...] = h
    a_ref[...] = jnp.dot(h, wa_ref[...], preferred_element_type=F32)
    b_ref[...] = jnp.dot(h, wb_ref[...], preferred_element_type=F32)


_prologue = pl.pallas_call(
    _prologue_body,
    grid=(NB,),
    in_specs=[
        pl.BlockSpec((BN, D), lambda i: (i, 0)),
        pl.BlockSpec((D, HID), lambda i: (0, 0)),
        pl.BlockSpec((1, HID), lambda i: (0, 0)),
        pl.BlockSpec((HID, HID), lambda i: (0, 0)),
        pl.BlockSpec((HID, HID), lambda i: (0, 0)),
    ],
    out_specs=[pl.BlockSpec((BN, HID), lambda i: (i, 0))] * 3,
    out_shape=[jax.ShapeDtypeStruct((NP, HID), F32)] * 3,
)


CR = E // 2
BTH = 6400
NBH = CR // BTH


def _cpre_body(lh_ref, rh_ref, wc_ref, cb_ref, c_ref):
    dn = (((0,), (0,)), ((), ()))
    t_lh = lax.dot_general(lh_ref[...], wc_ref[0], dn,
                           preferred_element_type=F32) + cb_ref[0]
    t_rh = lax.dot_general(rh_ref[...], wc_ref[0], dn,
                           preferred_element_type=F32) + cb_ref[0]
    c_ref[0] = jnp.concatenate([t_lh, t_rh], axis=1)


_cpre = pl.pallas_call(
    _cpre_body,
    grid=(3, NBH),
    in_specs=[
        pl.BlockSpec((ED, BTH), lambda l, e: (0, e)),
        pl.BlockSpec((ED, BTH), lambda l, e: (0, e + NBH)),
        pl.BlockSpec((1, ED, HID), lambda l, e: (l, 0, 0)),
        pl.BlockSpec((1, 1, HID), lambda l, e: (l, 0, 0)),
    ],
    out_specs=pl.BlockSpec((1, BTH, 128), lambda l, e: (l, e, 0)),
    out_shape=jax.ShapeDtypeStruct((3, CR, 128), F32),
)


RING = 2
SUP = (CH - 1) // RING
ZR = 64


def _make_edge_sc(layer, with_cnt):
    out_type = [jax.ShapeDtypeStruct((NC, NP, HID), F32)]
    if with_cnt:
        out_type.append(jax.ShapeDtypeStruct((NC, NP, CW), F32))
    scratch = [
        pltpu.VMEM((CH, K), jnp.int32),
        pltpu.VMEM((CH, K), jnp.int32),
        pltpu.VMEM((RING * K, HID), F32),
        pltpu.VMEM((RING * K, HID), F32),
        pltpu.VMEM((RING * K, HID), F32),
        pltpu.VMEM((RING * K, HID), F32),
        pltpu.VMEM((ZR, HID), F32),
        pltpu.VMEM_SHARED((NP, HID), F32),
        [pltpu.SemaphoreType.DMA] * RING,
        [pltpu.SemaphoreType.DMA] * RING,
        [pltpu.SemaphoreType.DMA] * RING,
        [pltpu.SemaphoreType.DMA] * RING,
    ]
    if with_cnt:
        scratch += [
            pltpu.VMEM((K, CW), F32),
            pltpu.VMEM((ZR, CW), F32),
            pltpu.VMEM_SHARED((NP, CW), F32),
            pltpu.SemaphoreType.DMA,
        ]

    def body(a_hbm, b_hbm, c_hbm, dst_hbm, src_hbm, s_out, *rest):
        if with_cnt:
            cnt_out, *rest = rest
        (dsti, srci, arow, brow, crow, res, zbuf, s_sh,
         sem_a, sem_b, sem_c, sem_s, *cextra) = rest
        cidx = lax.axis_index("c")
        sidx = lax.axis_index("s")
        wid = sidx * NC + cidx
        base = wid * EW
        in_lh = wid < NW // 2
        rbase = jnp.where(in_lh, wid, wid - NW // 2) * EW

        pltpu.sync_copy(dst_hbm.at[pl.ds(wid * CH, CH)], dsti)
        pltpu.sync_copy(src_hbm.at[pl.ds(wid * CH, CH)], srci)

        def start_loads(g, s):
            pltpu.async_copy(a_hbm.at[dsti.at[g]], arow.at[pl.ds(s * K, K)], sem_a[s])
            pltpu.async_copy(b_hbm.at[srci.at[g]], brow.at[pl.ds(s * K, K)], sem_b[s])
            crows = pl.ds(rbase + g * K, K)

            @pl.when(in_lh)
            def _():
                pltpu.async_copy(c_hbm.at[layer, crows, pl.ds(0, HID)],
                                 crow.at[pl.ds(s * K, K)], sem_c[s])

            @pl.when(jnp.logical_not(in_lh))
            def _():
                pltpu.async_copy(c_hbm.at[layer, crows, pl.ds(HID, HID)],
                                 crow.at[pl.ds(s * K, K)], sem_c[s])

        def wait_loads(g, s):
            pltpu.make_async_copy(a_hbm.at[dsti.at[g]], arow.at[pl.ds(s * K, K)], sem_a[s]).wait()
            pltpu.make_async_copy(b_hbm.at[srci.at[g]], brow.at[pl.ds(s * K, K)], sem_b[s]).wait()
            pltpu.make_async_copy(
                c_hbm.at[layer, pl.ds(rbase + g * K, K), pl.ds(0, HID)],
                crow.at[pl.ds(s * K, K)], sem_c[s]).wait()

        def scatter_wait(g, s):
            pltpu.make_async_copy(res.at[pl.ds(s * K, K)], s_sh.at[dsti.at[g]], sem_s[s]).wait()

        def compute(g, s):
            def erow(i, _):
                for j in range(HID // L):
                    sl = pl.ds(j * L, L)
                    res[s * K + i, sl] = jnp.maximum(
                        arow[s * K + i, sl] + brow[s * K + i, sl]
                        + crow[s * K + i, sl], 0.0)
                return 0
            lax.fori_loop(0, K, erow, 0)
            pltpu.async_copy(res.at[pl.ds(s * K, K)], s_sh.at[dsti.at[g]], sem_s[s], add=True)
            if with_cnt:
                pltpu.async_copy(cextra[0], cextra[2].at[dsti.at[g]],
                                 cextra[3], add=True)

        for s in range(RING - 1):
            start_loads(s, s)

        def zrow(i, _):
            for j in range(HID // L):
                zbuf[i, pl.ds(j * L, L)] = jnp.zeros((L,), F32)
            return 0
        lax.fori_loop(0, ZR, zrow, 0)
        for r in range(RW // ZR):
            pltpu.sync_copy(zbuf, s_sh.at[pl.ds(sidx * RW + r * ZR, ZR)])
        if with_cnt:
            ones, zcbuf, cnt_sh, sem_n = cextra
            def cinit(i, _):
                ones[i, pl.ds(0, L)] = jnp.ones((L,), F32)
                zcbuf[i % ZR, pl.ds(0, L)] = jnp.zeros((L,), F32)
                return 0
            lax.fori_loop(0, K, cinit, 0)
            for r in range(RW // ZR):
                pltpu.sync_copy(zcbuf, cnt_sh.at[pl.ds(sidx * RW + r * ZR, ZR)])
        plsc.subcore_barrier()

        def turn(sup, _):
            for s in range(RING):
                g = RING * sup + s

                @pl.when(sup >= 1)
                def _():
                    scatter_wait(g - RING, s)
                wait_loads(g, s)
                compute(g, s)
                nxt = g + RING - 1

                @pl.when(nxt < CH)
                def _():
                    start_loads(nxt, (s + RING - 1) % RING)
            return 0
        lax.fori_loop(0, SUP, turn, 0)

        g_last = RING * SUP
        scatter_wait(g_last - RING, 0)
        wait_loads(g_last, 0)
        compute(g_last, 0)
        for s in range(RING):
            scatter_wait(RING * (SUP - 1) + s if s else g_last, s)
        if with_cnt:
            def ndrain(g, _):
                pltpu.make_async_copy(cextra[0], cextra[2].at[dsti.at[g]],
                                      cextra[3]).wait()
                return 0
            lax.fori_loop(0, CH, ndrain, 0)

        plsc.subcore_barrier()
        for r in range(RW // ZR):
            rows = pl.ds(sidx * RW + r * ZR, ZR)
            pltpu.sync_copy(s_sh.at[rows], s_out.at[cidx, rows])
            if with_cnt:
                pltpu.sync_copy(cextra[2].at[rows], cnt_out.at[cidx, rows])

    return pl.kernel(
        body,
        out_type=out_type,
        mesh=plsc.VectorSubcoreMesh(core_axis_name="c", subcore_axis_name="s"),
        scratch_types=scratch,
        compiler_params=pltpu.CompilerParams(use_tc_tiling_on_sc=False),
    )


_edge_sc = [_make_edge_sc(0, True), _make_edge_sc(1, False), _make_edge_sc(2, False)]


def _node_body(h_ref, s_ref, cnt_ref, u1a_ref, m2u_ref, b2u_ref, u1b_ref,
               u2w_ref, u2b_ref, wa_ref, wb_ref, hn_ref, an_ref, bn_ref):
    h = h_ref[...]
    ss = s_ref[0] + s_ref[1]
    c2 = cnt_ref[0, :, :1] + cnt_ref[1, :, :1]
    t = (jnp.dot(h, u1a_ref[...], preferred_element_type=F32)
         + jnp.dot(ss, m2u_ref[...], preferred_element_type=F32)
         + c2 * b2u_ref[...] + u1b_ref[...])
    u = jnp.dot(jnp.maximum(t, 0.0), u2w_ref[...], preferred_element_type=F32) + u2b_ref[...]
    hn = h + u
    hn_ref[...] = hn
    an_ref[...] = jnp.dot(hn, wa_ref[...], preferred_element_type=F32)
    bn_ref[...] = jnp.dot(hn, wb_ref[...], preferred_element_type=F32)


_node_update = pl.pallas_call(
    _node_body,
    grid=(NB,),
    in_specs=[
        pl.BlockSpec((BN, HID), lambda i: (i, 0)),
        pl.BlockSpec((NC, BN, HID), lambda i: (0, i, 0)),
        pl.BlockSpec((NC, BN, CW), lambda i: (0, i, 0)),
    ] + [pl.BlockSpec((HID, HID), lambda i: (0, 0)),
         pl.BlockSpec((HID, HID), lambda i: (0, 0)),
         pl.BlockSpec((1, HID), lambda i: (0, 0)),
         pl.BlockSpec((1, HID), lambda i: (0, 0)),
         pl.BlockSpec((HID, HID), lambda i: (0, 0)),
         pl.BlockSpec((1, HID), lambda i: (0, 0)),
         pl.BlockSpec((HID, HID), lambda i: (0, 0)),
         pl.BlockSpec((HID, HID), lambda i: (0, 0))],
    out_specs=[pl.BlockSpec((BN, HID), lambda i: (i, 0))] * 3,
    out_shape=[jax.ShapeDtypeStruct((NP, HID), F32)] * 3,
)


def _final_body(h_ref, s_ref, cnt_ref, batch_ref, gf_ref,
                u1a_ref, m2u_ref, b2u_ref, u1b_ref, u2w_ref, u2b_ref,
                r1a_ref, gw_ref, rb1_ref, r2w_ref, r2b_ref, r3w_ref, r3b_ref,
                out_ref, psum, pcnt):
    i = pl.program_id(0)

    @pl.when(i == 0)
    def _():
        psum[...] = jnp.zeros_like(psum)
        pcnt[...] = jnp.zeros_like(pcnt)

    h = h_ref[...]
    ss = s_ref[0] + s_ref[1]
    c2 = cnt_ref[0, :, :1] + cnt_ref[1, :, :1]
    t = (jnp.dot(h, u1a_ref[...], preferred_element_type=F32)
         + jnp.dot(ss, m2u_ref[...], preferred_element_type=F32)
         + c2 * b2u_ref[...] + u1b_ref[...])
    u = jnp.dot(jnp.maximum(t, 0.0), u2w_ref[...], preferred_element_type=F32) + u2b_ref[...]
    hn = h + u
    oh = (batch_ref[...] == lax.broadcasted_iota(jnp.int32, (1, G), 1)).astype(F32)
    dn = (((0,), (0,)), ((), ()))
    psum[...] += lax.dot_general(oh, hn, dn, preferred_element_type=F32)
    pcnt[...] += lax.dot_general(oh, jnp.ones((BN, 1), F32), dn, preferred_element_type=F32)

    @pl.when(i == NB - 1)
    def _():
        pooled = psum[...] / jnp.maximum(pcnt[...], 1.0)
        t1 = (jnp.dot(pooled, r1a_ref[...], preferred_element_type=F32)
              + jnp.dot(gf_ref[...], gw_ref[...], preferred_element_type=F32)
              + rb1_ref[...])
        r1 = jnp.maximum(t1, 0.0)
        r2 = jnp.maximum(jnp.dot(r1, r2w_ref[...], preferred_element_type=F32) + r2b_ref[...], 0.0)
        out_ref[...] = jnp.dot(r2, r3w_ref[...], preferred_element_type=F32) + r3b_ref[...]


_final = pl.pallas_call(
    _final_body,
    grid=(NB,),
    in_specs=[
        pl.BlockSpec((BN, HID), lambda i: (i, 0)),
        pl.BlockSpec((NC, BN, HID), lambda i: (0, i, 0)),
        pl.BlockSpec((NC, BN, CW), lambda i: (0, i, 0)),
        pl.BlockSpec((BN, 1), lambda i: (i, 0)),
        pl.BlockSpec((G, 1), lambda i: (0, 0)),
    ] + [pl.BlockSpec((HID, HID), lambda i: (0, 0)),
         pl.BlockSpec((HID, HID), lambda i: (0, 0)),
         pl.BlockSpec((1, HID), lambda i: (0, 0)),
         pl.BlockSpec((1, HID), lambda i: (0, 0)),
         pl.BlockSpec((HID, HID), lambda i: (0, 0)),
         pl.BlockSpec((1, HID), lambda i: (0, 0)),
         pl.BlockSpec((HID, HID), lambda i: (0, 0)),
         pl.BlockSpec((1, HID), lambda i: (0, 0)),
         pl.BlockSpec((1, HID), lambda i: (0, 0)),
         pl.BlockSpec((HID, HID // 2), lambda i: (0, 0)),
         pl.BlockSpec((1, HID // 2), lambda i: (0, 0)),
         pl.BlockSpec((HID // 2, 1), lambda i: (0, 0)),
         pl.BlockSpec((1, 1), lambda i: (0, 0))],
    out_specs=pl.BlockSpec((G, 1), lambda i: (0, 0)),
    out_shape=jax.ShapeDtypeStruct((G, 1), F32),
    scratch_shapes=[pltpu.VMEM((G, HID), F32), pltpu.VMEM((G, 1), F32)],
)


def kernel(x, edge_index, edge_attr, batch, global_feature, params):
    p = params
    layers = p['layers']

    xp = jnp.zeros((NP, D), F32).at[:N].set(x)
    batch_p = jnp.full((NP, 1), G, jnp.int32).at[:N, 0].set(batch.astype(jnp.int32))
    src = edge_index[0].astype(jnp.int32).reshape(NW * CH, K)
    dst = edge_index[1].astype(jnp.int32).reshape(NW * CH, K)

    W1a = [lp['m1_w'][:HID] for lp in layers]
    W1b = [lp['m1_w'][HID:2 * HID] for lp in layers]
    wc_all = jnp.stack([p['edge_w'] @ lp['m1_w'][2 * HID:] for lp in layers])
    cb_all = jnp.stack([(p['edge_b'] @ lp['m1_w'][2 * HID:] + lp['m1_b'])[None]
                        for lp in layers])
    ea_t = edge_attr.T
    U1a = [lp['u1_w'][:HID] for lp in layers]
    M2U = [lp['m2_w'] @ lp['u1_w'][HID:] for lp in layers]
    b2u = [(lp['m2_b'] @ lp['u1_w'][HID:])[None] for lp in layers]
    u1b = [lp['u1_b'][None] for lp in layers]
    R1b = p['r1_w'][HID:]
    gw = p['glob_w'] @ R1b
    rb1 = (p['r1_b'] + p['glob_b'] @ R1b)[None]

    h, A, B = _prologue(xp, p['node_w'], p['node_b'][None], W1a[0], W1b[0])
    C = _cpre(ea_t, ea_t, wc_all, cb_all)

    S, cnt = _edge_sc[0](A, B, C, dst, src)
    for l in range(2):
        h, A, B = _node_update(h, S, cnt, U1a[l], M2U[l], b2u[l], u1b[l],
                               layers[l]['u2_w'], layers[l]['u2_b'][None],
                               W1a[l + 1], W1b[l + 1])
        (S,) = _edge_sc[l + 1](A, B, C, dst, src)

    return _final(h, S, cnt, batch_p, global_feature,
                  U1a[2], M2U[2], b2u[2], u1b[2],
                  layers[2]['u2_w'], layers[2]['u2_b'][None],
                  p['r1_w'][:HID], gw, rb1,
                  p['r2_w'], p['r2_b'][None], p['r3_w'], p['r3_b'][None])
